# baseline (speedup 1.0000x reference)
import functools

import jax
import jax.numpy as jnp
from jax import lax
from jax.experimental import pallas as pl
from jax.experimental.pallas import tpu as pltpu
from jax.experimental.pallas import tpu_sc as plsc

N = 10000
E = 160000
D = 256
NREL = 16
NPAD = 10240
NC, NS, L = 2, 16, 16
NTILE = NC * NS
ROWS = NPAD // NTILE
CH = 1600
NCHUNK = E // CH
KB = 16
OVFL = 4 * KB
QCAP = CH + OVFL + 2 * L


def _mm_body(x_ref, w_ref, a_ref, r_ref, ar_ref, h_ref, aux_ref, ra_ref):
    h = jnp.dot(x_ref[...], w_ref[...], preferred_element_type=jnp.float32)
    h_ref[...] = h
    aux_ref[...] = jnp.dot(h, a_ref[...], preferred_element_type=jnp.float32)
    ra_ref[...] = jnp.dot(r_ref[...], ar_ref[...],
                          preferred_element_type=jnp.float32)


def _mm(x_pad, W, A, R, Ar):
    BM = 1024
    return pl.pallas_call(
        _mm_body,
        grid=(NPAD // BM,),
        in_specs=[
            pl.BlockSpec((BM, D), lambda i: (i, 0)),
            pl.BlockSpec((D, D), lambda i: (0, 0)),
            pl.BlockSpec((D, 128), lambda i: (0, 0)),
            pl.BlockSpec((NREL, D), lambda i: (0, 0)),
            pl.BlockSpec((D, 128), lambda i: (0, 0)),
        ],
        out_specs=[
            pl.BlockSpec((BM, D), lambda i: (i, 0)),
            pl.BlockSpec((BM, 128), lambda i: (i, 0)),
            pl.BlockSpec((NREL, 128), lambda i: (0, 0)),
        ],
        out_shape=[
            jax.ShapeDtypeStruct((NPAD, D), jnp.float32),
            jax.ShapeDtypeStruct((NPAD, 128), jnp.float32),
            jax.ShapeDtypeStruct((NREL, 128), jnp.float32),
        ],
    )(x_pad, W, A, R, Ar)


def _ls_body(x_ref, o_ref):
    y = x_ref[...]
    m = jnp.max(y, axis=1, keepdims=True)
    z = y - m
    lse = jnp.log(jnp.sum(jnp.exp(z), axis=1, keepdims=True))
    o_ref[...] = z - lse


def _log_softmax(x):
    BM = 1024
    return pl.pallas_call(
        _ls_body,
        grid=(NPAD // BM,),
        in_specs=[pl.BlockSpec((BM, D), lambda i: (i, 0))],
        out_specs=pl.BlockSpec((BM, D), lambda i: (i, 0)),
        out_shape=jax.ShapeDtypeStruct((NPAD, D), jnp.float32),
    )(x)


def _make_sc_layer(do_relu):
    mesh = plsc.VectorSubcoreMesh(core_axis_name="c", subcore_axis_name="s")

    @functools.partial(
        pl.kernel,
        out_type=[jax.ShapeDtypeStruct((NPAD, D), jnp.float32),
                  jax.ShapeDtypeStruct((E,), jnp.float32)],
        mesh=mesh,
        scratch_types=[
            pltpu.VMEM((NPAD,), jnp.float32),
            pltpu.VMEM((NPAD,), jnp.float32),
            pltpu.VMEM((NREL,), jnp.float32),
            pltpu.VMEM((NREL, D), jnp.float32),
            pltpu.VMEM((D,), jnp.float32),
            pltpu.VMEM((CH,), jnp.int32),
            pltpu.VMEM((CH,), jnp.int32),
            pltpu.VMEM((CH,), jnp.float32),
            pltpu.VMEM((CH,), jnp.float32),
            pltpu.VMEM((CH,), jnp.float32),
            pltpu.VMEM((CH,), jnp.int32),
            pltpu.VMEM((QCAP,), jnp.int32),
            pltpu.VMEM((QCAP,), jnp.float32),
            pltpu.VMEM((KB,), jnp.int32),
            pltpu.VMEM((KB,), jnp.int32),
            pltpu.VMEM((2 * KB,), jnp.int32),
            pltpu.VMEM((2 * KB,), jnp.int32),
            pltpu.VMEM((2 * KB,), jnp.float32),
            pltpu.VMEM((KB, D), jnp.float32),
            pltpu.VMEM((KB, D), jnp.float32),
            pltpu.VMEM((ROWS, D), jnp.float32),
            pltpu.VMEM((ROWS,), jnp.float32),
            pltpu.VMEM((D,), jnp.float32),
            pltpu.VMEM_SHARED((NPAD,), jnp.float32),
            pltpu.SemaphoreType.DMA,
            pltpu.SemaphoreType.DMA,
            pltpu.SemaphoreType.DMA,
            pltpu.SemaphoreType.DMA,
        ],
        compiler_params=pltpu.CompilerParams(needs_layout_passes=False),
    )
    def sc_layer(hs_hbm, hd_hbm, ra_hbm, R_hbm, epk_hbm, bias_hbm, h_hbm,
                 out_hbm, p_hbm,
                 hs_v, hd_v, ra_v, R_v, bias_v, epk_a, epk_b, p_ca, p_cb,
                 p_b, d_b, qmeta, qa, qs_a, qs_b2, qt_b, qd_b, qa_b,
                 hrow_a, hrow_b, out_blk, den_v,
                 zflat, den_spmem, sem_ca, sem_cb, sem_ga, sem_gb):
        cid = lax.axis_index("c")
        sid = lax.axis_index("s")
        wid = cid * NS + sid
        lo = wid * ROWS

        zero_f = jnp.zeros((L,), jnp.float32)
        iota = lax.iota(jnp.int32, L)

        pltpu.sync_copy(hs_hbm, hs_v)
        pltpu.sync_copy(hd_hbm, hd_v)
        pltpu.sync_copy(ra_hbm, ra_v)
        pltpu.sync_copy(R_hbm, R_v)
        pltpu.sync_copy(bias_hbm, bias_v)

        for c in range(D // L):
            zflat[pl.ds(c * L, L)] = zero_f

        def z_body(r, _):
            for c in range(D // L):
                out_blk[r, pl.ds(c * L, L)] = zero_f
            return _

        lax.fori_loop(0, ROWS, z_body, 0)

        @pl.when(sid == 0)
        def _zero_den():
            for j in range(NPAD // D):
                pltpu.sync_copy(zflat, den_spmem.at[pl.ds(j * D, D)])

        plsc.subcore_barrier()

        nch1 = (NCHUNK // NS) + jnp.where(sid < (NCHUNK % NS), 1, 0)

        def p1_chunk(j, _):
            c = sid + j * NS
            pltpu.sync_copy(epk_hbm.at[pl.ds(c * CH, CH)], epk_a)

            def p1_it(i, __):
                w = epk_a[pl.ds(i * L, L)]
                s = w & 0x3FFF
                d = (w >> 14) & 0x3FFF
                t = (w >> 28) & 0xF
                lg = (plsc.load_gather(hs_v, [s])
                      + plsc.load_gather(hd_v, [d])
                      + plsc.load_gather(ra_v, [t]))
                lg = jnp.maximum(lg, 0.2 * lg)
                p = jnp.exp(jnp.minimum(lg, 60.0))
                p_b[pl.ds(i * L, L)] = p
                d_b[pl.ds(i * L, L)] = d
                return __

            lax.fori_loop(0, CH // L, p1_it, 0)
            pltpu.sync_copy(p_b, den_spmem.at[d_b], add=True)
            pltpu.sync_copy(p_b, p_hbm.at[pl.ds(c * CH, CH)])
            return _

        lax.fori_loop(0, nch1, p1_chunk, 0)

        plsc.subcore_barrier()

        pltpu.sync_copy(den_spmem.at[pl.ds(lo, ROWS)], den_v)

        def prep_idx(b, qs_x):
            meta = qmeta[pl.ds(b * KB, L)]
            qs_x[...] = meta & 0x3FFF

        def start_gather(qs_x, hrow_x, sem_x):
            pltpu.async_copy(h_hbm.at[qs_x], hrow_x, sem_x)

        def wait_gather(qs_x, hrow_x, sem_x):
            pltpu.make_async_copy(h_hbm.at[qs_x], hrow_x, sem_x).wait()

        def compute_batch(b, hrow_x):
            off = b * KB
            meta = qmeta[pl.ds(off, L)]
            qt_b[pl.ds(0, L)] = (meta >> 14) & 0xF
            qd_b[pl.ds(0, L)] = meta >> 18
            qa_b[pl.ds(0, L)] = qa[pl.ds(off, L)]

            def ej(j, _):
                aj = qa_b[pl.ds(j, L)][0]
                tj = qt_b[pl.ds(j, L)][0]
                dj = qd_b[pl.ds(j, L)][0]
                for c in range(D // L):
                    plsc.addupdate(
                        out_blk.at[dj, pl.ds(c * L, L)],
                        aj * hrow_x[j, pl.ds(c * L, L)]
                        * R_v[tj, pl.ds(c * L, L)])
                return _

            lax.fori_loop(0, KB, ej, 0)

        def start_first_batches(nf):
            @pl.when(nf > 0)
            def _():
                prep_idx(0, qs_a)
                start_gather(qs_a, hrow_a, sem_ga)

            @pl.when(nf > 1)
            def _():
                prep_idx(1, qs_b2)
                start_gather(qs_b2, hrow_b, sem_gb)

        def drain_batches(nf):
            def pair(j, zz):
                b0 = 2 * j
                b1 = b0 + 1

                @pl.when(b0 < nf)
                def _():
                    wait_gather(qs_a, hrow_a, sem_ga)
                    compute_batch(b0, hrow_a)

                    @pl.when(b0 + 2 < nf)
                    def _():
                        prep_idx(b0 + 2, qs_a)
                        start_gather(qs_a, hrow_a, sem_ga)

                @pl.when(b1 < nf)
                def _():
                    wait_gather(qs_b2, hrow_b, sem_gb)
                    compute_batch(b1, hrow_b)

                    @pl.when(b1 + 2 < nf)
                    def _():
                        prep_idx(b1 + 2, qs_b2)
                        start_gather(qs_b2, hrow_b, sem_gb)

                return zz

            lax.fori_loop(0, (nf + 1) // 2, pair, 0)

        def slide_queue(qp, nf):
            rem = qp - nf * KB

            @pl.when(nf > 0)
            def _():
                def mv(k, _):
                    vm = qmeta[pl.ds(nf * KB + k * L, L)]
                    va = qa[pl.ds(nf * KB + k * L, L)]
                    qmeta[pl.ds(k * L, L)] = vm
                    qa[pl.ds(k * L, L)] = va
                    return _

                lax.fori_loop(0, (rem + L - 1) // L, mv, 0)

            return rem

        def scan_chunk(epk_x, p_cx, qp):
            def p2_it(i, qp):
                w = epk_x[pl.ds(i * L, L)]
                s = w & 0x3FFF
                d = (w >> 14) & 0x3FFF
                t = (w >> 28) & 0xF
                dloc = d - lo
                m = (dloc >= 0) & (dloc < ROWS)
                dlc = jnp.clip(dloc, 0, ROWS - 1)
                p = p_cx[pl.ds(i * L, L)]
                den_g = plsc.load_gather(den_v, [dlc])
                a = p / (den_g + 1e-16)
                pk = s | (t << 14) | (dlc << 18)
                plsc.store_compressed(qmeta.at[pl.ds(qp, L)], pk, mask=m)
                plsc.store_compressed(qa.at[pl.ds(qp, L)], a, mask=m)
                cnt = plsc.all_reduce_population_count(m)
                return qp + cnt[0]

            return lax.fori_loop(0, CH // L, p2_it, qp)

        def start_chunk(c, epk_x, p_cx, sem_x):
            pltpu.async_copy(epk_hbm.at[pl.ds(c * CH, CH)], epk_x, sem_x)
            pltpu.async_copy(p_hbm.at[pl.ds(c * CH, CH)], p_cx, sem_x)

        def wait_chunk(epk_x, p_cx, sem_x):
            pltpu.make_async_copy(epk_hbm.at[pl.ds(0, CH)],
                                  epk_x, sem_x).wait()
            pltpu.make_async_copy(p_hbm.at[pl.ds(0, CH)],
                                  p_cx, sem_x).wait()

        def drain_inflight(np):
            @pl.when(np > 0)
            def _():
                wait_gather(qs_a, hrow_a, sem_ga)
                compute_batch(0, hrow_a)

            @pl.when(np > 1)
            def _():
                wait_gather(qs_b2, hrow_b, sem_gb)
                compute_batch(1, hrow_b)

        def handle_chunk(c, epk_x, p_cx, sem_x, carry):
            qp, np = carry
            wait_chunk(epk_x, p_cx, sem_x)
            qp = scan_chunk(epk_x, p_cx, qp)

            @pl.when(c + 2 < NCHUNK)
            def _():
                start_chunk(c + 2, epk_x, p_cx, sem_x)

            drain_inflight(np)
            qp = slide_queue(qp, np)

            nf_x = jnp.where(qp >= OVFL, qp // KB, 0)
            start_first_batches(nf_x)
            drain_batches(nf_x)
            qp = slide_queue(qp, nf_x)

            np = jnp.minimum(qp // KB, 2)
            start_first_batches(np)
            return qp, np

        start_chunk(0, epk_a, p_ca, sem_ca)
        start_chunk(1, epk_b, p_cb, sem_cb)

        def p2_pair(i, carry):
            carry = handle_chunk(2 * i, epk_a, p_ca, sem_ca, carry)
            carry = handle_chunk(2 * i + 1, epk_b, p_cb, sem_cb, carry)
            return carry

        qp, np = lax.fori_loop(0, NCHUNK // 2, p2_pair, (0, 0))

        drain_inflight(np)
        qp = slide_queue(qp, np)
        qmeta[pl.ds(qp, L)] = jnp.zeros((L,), jnp.int32)
        qa[pl.ds(qp, L)] = zero_f
        nfin = (qp + KB - 1) // KB
        start_first_batches(nfin)
        drain_batches(nfin)

        def o_body(r, _):
            for c in range(D // L):
                v = out_blk[r, pl.ds(c * L, L)] + bias_v[pl.ds(c * L, L)]
                if do_relu:
                    v = jnp.maximum(v, 0.0)
                out_blk[r, pl.ds(c * L, L)] = v
            return _

        lax.fori_loop(0, ROWS, o_body, 0)
        pltpu.sync_copy(out_blk, out_hbm.at[pl.ds(lo, ROWS)])

    return sc_layer


_sc_layer_relu = _make_sc_layer(True)
_sc_layer_plain = _make_sc_layer(False)


def _layer(x_pad, epk, W, R, a_s, a_d, a_r, b, do_relu):
    A = jnp.zeros((D, 128), jnp.float32)
    A = A.at[:, 0].set(a_s).at[:, 1].set(a_d)
    Ar = jnp.zeros((D, 128), jnp.float32).at[:, 0].set(a_r)
    h, aux, raux = _mm(x_pad, W, A, R, Ar)
    hs = aux[:, 0]
    hd = aux[:, 1]
    ra = raux[:NREL, 0]
    fn = _sc_layer_relu if do_relu else _sc_layer_plain
    out, _p = fn(hs, hd, ra, R, epk, b, h)
    return out


def kernel(x, edge, edge_type, W1, R1, a1_src, a1_dst, a1_rel, b1,
           W2, R2, a2_src, a2_dst, a2_rel, b2):
    epk = edge[0] | (edge[1] << 14) | (edge_type << 28)
    x_pad = jnp.zeros((NPAD, D), jnp.float32).at[:N].set(x)
    h1 = _layer(x_pad, epk, W1, R1, a1_src, a1_dst, a1_rel, b1, True)
    h2 = _layer(h1, epk, W2, R2, a2_src, a2_dst, a2_rel, b2, False)
    out = _log_softmax(h2)
    return out[:N]


# speedup vs baseline: 1.0493x; 1.0493x over previous
import functools

import jax
import jax.numpy as jnp
from jax import lax
from jax.experimental import pallas as pl
from jax.experimental.pallas import tpu as pltpu
from jax.experimental.pallas import tpu_sc as plsc

N = 10000
E = 160000
D = 256
NREL = 16
NPAD = 10240
NC, NS, L = 2, 16, 16
NTILE = NC * NS
ROWS = NPAD // NTILE
CH = 800
NCHUNK = E // CH
KB = 16
OVFL = 6 * KB
QCAP = CH + OVFL + 2 * L


def _mm_body(x_ref, w_ref, a_ref, r_ref, ar_ref, h_ref, aux_ref, ra_ref):
    h = jnp.dot(x_ref[...], w_ref[...], preferred_element_type=jnp.float32)
    h_ref[...] = h
    aux_ref[...] = jnp.dot(h, a_ref[...], preferred_element_type=jnp.float32)
    ra_ref[...] = jnp.dot(r_ref[...], ar_ref[...],
                          preferred_element_type=jnp.float32)


def _mm(x_pad, W, A, R, Ar):
    BM = 1024
    return pl.pallas_call(
        _mm_body,
        grid=(NPAD // BM,),
        in_specs=[
            pl.BlockSpec((BM, D), lambda i: (i, 0)),
            pl.BlockSpec((D, D), lambda i: (0, 0)),
            pl.BlockSpec((D, 128), lambda i: (0, 0)),
            pl.BlockSpec((NREL, D), lambda i: (0, 0)),
            pl.BlockSpec((D, 128), lambda i: (0, 0)),
        ],
        out_specs=[
            pl.BlockSpec((BM, D), lambda i: (i, 0)),
            pl.BlockSpec((BM, 128), lambda i: (i, 0)),
            pl.BlockSpec((NREL, 128), lambda i: (0, 0)),
        ],
        out_shape=[
            jax.ShapeDtypeStruct((NPAD, D), jnp.float32),
            jax.ShapeDtypeStruct((NPAD, 128), jnp.float32),
            jax.ShapeDtypeStruct((NREL, 128), jnp.float32),
        ],
    )(x_pad, W, A, R, Ar)


def _ls_body(x_ref, o_ref):
    y = x_ref[...]
    m = jnp.max(y, axis=1, keepdims=True)
    z = y - m
    lse = jnp.log(jnp.sum(jnp.exp(z), axis=1, keepdims=True))
    o_ref[...] = z - lse


def _log_softmax(x):
    BM = 1024
    return pl.pallas_call(
        _ls_body,
        grid=(NPAD // BM,),
        in_specs=[pl.BlockSpec((BM, D), lambda i: (i, 0))],
        out_specs=pl.BlockSpec((BM, D), lambda i: (i, 0)),
        out_shape=jax.ShapeDtypeStruct((NPAD, D), jnp.float32),
    )(x)


def _make_sc_layer(do_relu):
    mesh = plsc.VectorSubcoreMesh(core_axis_name="c", subcore_axis_name="s")

    @functools.partial(
        pl.kernel,
        out_type=[jax.ShapeDtypeStruct((NPAD, D), jnp.float32),
                  jax.ShapeDtypeStruct((E,), jnp.float32)],
        mesh=mesh,
        scratch_types=[
            pltpu.VMEM((NPAD,), jnp.float32),
            pltpu.VMEM((NPAD,), jnp.float32),
            pltpu.VMEM((NREL,), jnp.float32),
            pltpu.VMEM((NREL, D), jnp.float32),
            pltpu.VMEM((D,), jnp.float32),
            pltpu.VMEM((CH,), jnp.int32),
            pltpu.VMEM((CH,), jnp.int32),
            pltpu.VMEM((CH,), jnp.float32),
            pltpu.VMEM((CH,), jnp.float32),
            pltpu.VMEM((CH,), jnp.float32),
            pltpu.VMEM((CH,), jnp.int32),
            pltpu.VMEM((QCAP,), jnp.int32),
            pltpu.VMEM((QCAP,), jnp.float32),
            pltpu.VMEM((KB,), jnp.int32),
            pltpu.VMEM((KB,), jnp.int32),
            pltpu.VMEM((2 * KB,), jnp.int32),
            pltpu.VMEM((2 * KB,), jnp.int32),
            pltpu.VMEM((2 * KB,), jnp.float32),
            pltpu.VMEM((KB, D), jnp.float32),
            pltpu.VMEM((KB, D), jnp.float32),
            pltpu.VMEM((ROWS, D), jnp.float32),
            pltpu.VMEM((ROWS,), jnp.float32),
            pltpu.VMEM((D,), jnp.float32),
            pltpu.VMEM_SHARED((NPAD,), jnp.float32),
            pltpu.SemaphoreType.DMA,
            pltpu.SemaphoreType.DMA,
            pltpu.SemaphoreType.DMA,
            pltpu.SemaphoreType.DMA,
        ],
        compiler_params=pltpu.CompilerParams(needs_layout_passes=False),
    )
    def sc_layer(hs_hbm, hd_hbm, ra_hbm, R_hbm, epk_hbm, bias_hbm, h_hbm,
                 out_hbm, p_hbm,
                 hs_v, hd_v, ra_v, R_v, bias_v, epk_a, epk_b, p_ca, p_cb,
                 p_b, d_b, qmeta, qa, qs_a, qs_b2, qt_b, qd_b, qa_b,
                 hrow_a, hrow_b, out_blk, den_v,
                 zflat, den_spmem, sem_ca, sem_cb, sem_ga, sem_gb):
        cid = lax.axis_index("c")
        sid = lax.axis_index("s")
        wid = cid * NS + sid
        lo = wid * ROWS

        zero_f = jnp.zeros((L,), jnp.float32)
        iota = lax.iota(jnp.int32, L)

        pltpu.sync_copy(hs_hbm, hs_v)
        pltpu.sync_copy(hd_hbm, hd_v)
        pltpu.sync_copy(ra_hbm, ra_v)
        pltpu.sync_copy(R_hbm, R_v)
        pltpu.sync_copy(bias_hbm, bias_v)

        for c in range(D // L):
            zflat[pl.ds(c * L, L)] = zero_f

        def z_body(r, _):
            for c in range(D // L):
                out_blk[r, pl.ds(c * L, L)] = zero_f
            return _

        lax.fori_loop(0, ROWS, z_body, 0)

        @pl.when(sid == 0)
        def _zero_den():
            for j in range(NPAD // D):
                pltpu.sync_copy(zflat, den_spmem.at[pl.ds(j * D, D)])

        plsc.subcore_barrier()

        nch1 = (NCHUNK // NS) + jnp.where(sid < (NCHUNK % NS), 1, 0)

        def p1_chunk(j, _):
            c = sid + j * NS
            pltpu.sync_copy(epk_hbm.at[pl.ds(c * CH, CH)], epk_a)

            def p1_it(i, __):
                w = epk_a[pl.ds(i * L, L)]
                s = w & 0x3FFF
                d = (w >> 14) & 0x3FFF
                t = (w >> 28) & 0xF
                lg = (plsc.load_gather(hs_v, [s])
                      + plsc.load_gather(hd_v, [d])
                      + plsc.load_gather(ra_v, [t]))
                lg = jnp.maximum(lg, 0.2 * lg)
                p = jnp.exp(jnp.minimum(lg, 60.0))
                p_b[pl.ds(i * L, L)] = p
                d_b[pl.ds(i * L, L)] = d
                return __

            lax.fori_loop(0, CH // L, p1_it, 0)
            pltpu.sync_copy(p_b, den_spmem.at[d_b], add=True)
            pltpu.sync_copy(p_b, p_hbm.at[pl.ds(c * CH, CH)])
            return _

        lax.fori_loop(0, nch1, p1_chunk, 0)

        plsc.subcore_barrier()

        pltpu.sync_copy(den_spmem.at[pl.ds(lo, ROWS)], den_v)

        def prep_idx(b, qs_x):
            meta = qmeta[pl.ds(b * KB, L)]
            qs_x[...] = meta & 0x3FFF

        def start_gather(qs_x, hrow_x, sem_x):
            pltpu.async_copy(h_hbm.at[qs_x], hrow_x, sem_x)

        def wait_gather(qs_x, hrow_x, sem_x):
            pltpu.make_async_copy(h_hbm.at[qs_x], hrow_x, sem_x).wait()

        def compute_batch(b, hrow_x):
            off = b * KB
            meta = qmeta[pl.ds(off, L)]
            qt_b[pl.ds(0, L)] = (meta >> 14) & 0xF
            qd_b[pl.ds(0, L)] = meta >> 18
            qa_b[pl.ds(0, L)] = qa[pl.ds(off, L)]

            def ej(j, _):
                aj = qa_b[pl.ds(j, L)][0]
                tj = qt_b[pl.ds(j, L)][0]
                dj = qd_b[pl.ds(j, L)][0]
                for c in range(D // L):
                    plsc.addupdate(
                        out_blk.at[dj, pl.ds(c * L, L)],
                        aj * hrow_x[j, pl.ds(c * L, L)]
                        * R_v[tj, pl.ds(c * L, L)])
                return _

            lax.fori_loop(0, KB, ej, 0)

        def start_first_batches(nf):
            @pl.when(nf > 0)
            def _():
                prep_idx(0, qs_a)
                start_gather(qs_a, hrow_a, sem_ga)

            @pl.when(nf > 1)
            def _():
                prep_idx(1, qs_b2)
                start_gather(qs_b2, hrow_b, sem_gb)

        def drain_batches(nf):
            def pair(j, zz):
                b0 = 2 * j
                b1 = b0 + 1

                @pl.when(b0 < nf)
                def _():
                    wait_gather(qs_a, hrow_a, sem_ga)
                    compute_batch(b0, hrow_a)

                    @pl.when(b0 + 2 < nf)
                    def _():
                        prep_idx(b0 + 2, qs_a)
                        start_gather(qs_a, hrow_a, sem_ga)

                @pl.when(b1 < nf)
                def _():
                    wait_gather(qs_b2, hrow_b, sem_gb)
                    compute_batch(b1, hrow_b)

                    @pl.when(b1 + 2 < nf)
                    def _():
                        prep_idx(b1 + 2, qs_b2)
                        start_gather(qs_b2, hrow_b, sem_gb)

                return zz

            lax.fori_loop(0, (nf + 1) // 2, pair, 0)

        def slide_queue(qp, nf):
            rem = qp - nf * KB

            @pl.when(nf > 0)
            def _():
                def mv(k, _):
                    vm = qmeta[pl.ds(nf * KB + k * L, L)]
                    va = qa[pl.ds(nf * KB + k * L, L)]
                    qmeta[pl.ds(k * L, L)] = vm
                    qa[pl.ds(k * L, L)] = va
                    return _

                lax.fori_loop(0, (rem + L - 1) // L, mv, 0)

            return rem

        def scan_chunk(epk_x, p_cx, qp):
            def p2_it(i, qp):
                w = epk_x[pl.ds(i * L, L)]
                s = w & 0x3FFF
                d = (w >> 14) & 0x3FFF
                t = (w >> 28) & 0xF
                dloc = d - lo
                m = (dloc >= 0) & (dloc < ROWS)
                dlc = jnp.clip(dloc, 0, ROWS - 1)
                p = p_cx[pl.ds(i * L, L)]
                den_g = plsc.load_gather(den_v, [dlc])
                a = p / (den_g + 1e-16)
                pk = s | (t << 14) | (dlc << 18)
                plsc.store_compressed(qmeta.at[pl.ds(qp, L)], pk, mask=m)
                plsc.store_compressed(qa.at[pl.ds(qp, L)], a, mask=m)
                cnt = plsc.all_reduce_population_count(m)
                return qp + cnt[0]

            return lax.fori_loop(0, CH // L, p2_it, qp)

        def start_chunk(c, epk_x, p_cx, sem_x):
            pltpu.async_copy(epk_hbm.at[pl.ds(c * CH, CH)], epk_x, sem_x)
            pltpu.async_copy(p_hbm.at[pl.ds(c * CH, CH)], p_cx, sem_x)

        def wait_chunk(epk_x, p_cx, sem_x):
            pltpu.make_async_copy(epk_hbm.at[pl.ds(0, CH)],
                                  epk_x, sem_x).wait()
            pltpu.make_async_copy(p_hbm.at[pl.ds(0, CH)],
                                  p_cx, sem_x).wait()

        def drain_inflight(np):
            @pl.when(np > 0)
            def _():
                wait_gather(qs_a, hrow_a, sem_ga)
                compute_batch(0, hrow_a)

            @pl.when(np > 1)
            def _():
                wait_gather(qs_b2, hrow_b, sem_gb)
                compute_batch(1, hrow_b)

        def handle_chunk(c, epk_x, p_cx, sem_x, carry):
            qp, np = carry
            wait_chunk(epk_x, p_cx, sem_x)
            qp = scan_chunk(epk_x, p_cx, qp)

            @pl.when(c + 2 < NCHUNK)
            def _():
                start_chunk(c + 2, epk_x, p_cx, sem_x)

            drain_inflight(np)
            qp = slide_queue(qp, np)

            nf_x = jnp.where(qp >= OVFL, qp // KB, 0)
            start_first_batches(nf_x)
            drain_batches(nf_x)
            qp = slide_queue(qp, nf_x)

            np = jnp.minimum(qp // KB, 2)
            start_first_batches(np)
            return qp, np

        start_chunk(0, epk_a, p_ca, sem_ca)
        start_chunk(1, epk_b, p_cb, sem_cb)

        def p2_pair(i, carry):
            carry = handle_chunk(2 * i, epk_a, p_ca, sem_ca, carry)
            carry = handle_chunk(2 * i + 1, epk_b, p_cb, sem_cb, carry)
            return carry

        qp, np = lax.fori_loop(0, NCHUNK // 2, p2_pair, (0, 0))

        drain_inflight(np)
        qp = slide_queue(qp, np)
        qmeta[pl.ds(qp, L)] = jnp.zeros((L,), jnp.int32)
        qa[pl.ds(qp, L)] = zero_f
        nfin = (qp + KB - 1) // KB
        start_first_batches(nfin)
        drain_batches(nfin)

        def o_body(r, _):
            for c in range(D // L):
                v = out_blk[r, pl.ds(c * L, L)] + bias_v[pl.ds(c * L, L)]
                if do_relu:
                    v = jnp.maximum(v, 0.0)
                out_blk[r, pl.ds(c * L, L)] = v
            return _

        lax.fori_loop(0, ROWS, o_body, 0)
        pltpu.sync_copy(out_blk, out_hbm.at[pl.ds(lo, ROWS)])

    return sc_layer


_sc_layer_relu = _make_sc_layer(True)
_sc_layer_plain = _make_sc_layer(False)


def _layer(x_pad, epk, W, R, a_s, a_d, a_r, b, do_relu):
    A = jnp.zeros((D, 128), jnp.float32)
    A = A.at[:, 0].set(a_s).at[:, 1].set(a_d)
    Ar = jnp.zeros((D, 128), jnp.float32).at[:, 0].set(a_r)
    h, aux, raux = _mm(x_pad, W, A, R, Ar)
    hs = aux[:, 0]
    hd = aux[:, 1]
    ra = raux[:NREL, 0]
    fn = _sc_layer_relu if do_relu else _sc_layer_plain
    out, _p = fn(hs, hd, ra, R, epk, b, h)
    return out


def kernel(x, edge, edge_type, W1, R1, a1_src, a1_dst, a1_rel, b1,
           W2, R2, a2_src, a2_dst, a2_rel, b2):
    epk = edge[0] | (edge[1] << 14) | (edge_type << 28)
    x_pad = jnp.zeros((NPAD, D), jnp.float32).at[:N].set(x)
    h1 = _layer(x_pad, epk, W1, R1, a1_src, a1_dst, a1_rel, b1, True)
    h2 = _layer(h1, epk, W2, R2, a2_src, a2_dst, a2_rel, b2, False)
    out = _log_softmax(h2)
    return out[:N]


# speedup vs baseline: 1.0685x; 1.0183x over previous
import functools

import jax
import jax.numpy as jnp
from jax import lax
from jax.experimental import pallas as pl
from jax.experimental.pallas import tpu as pltpu
from jax.experimental.pallas import tpu_sc as plsc

N = 10000
E = 160000
D = 256
NREL = 16
NPAD = 10240
NC, NS, L = 2, 16, 16
NTILE = NC * NS
ROWS = NPAD // NTILE
CH = 800
NCHUNK = E // CH
KB = 16
OVFL = 6 * KB
QCAP = CH + OVFL + 2 * L


def _mm_body(x_ref, w_ref, a_ref, r_ref, ar_ref, h_ref, aux_ref, ra_ref):
    h = jnp.dot(x_ref[...], w_ref[...], preferred_element_type=jnp.float32)
    h_ref[...] = h
    aux_ref[...] = jnp.dot(h, a_ref[...], preferred_element_type=jnp.float32)
    ra_ref[...] = jnp.dot(r_ref[...], ar_ref[...],
                          preferred_element_type=jnp.float32)


def _mm(x_pad, W, A, R, Ar):
    BM = 1024
    return pl.pallas_call(
        _mm_body,
        grid=(NPAD // BM,),
        in_specs=[
            pl.BlockSpec((BM, D), lambda i: (i, 0)),
            pl.BlockSpec((D, D), lambda i: (0, 0)),
            pl.BlockSpec((D, 128), lambda i: (0, 0)),
            pl.BlockSpec((NREL, D), lambda i: (0, 0)),
            pl.BlockSpec((D, 128), lambda i: (0, 0)),
        ],
        out_specs=[
            pl.BlockSpec((BM, D), lambda i: (i, 0)),
            pl.BlockSpec((BM, 128), lambda i: (i, 0)),
            pl.BlockSpec((NREL, 128), lambda i: (0, 0)),
        ],
        out_shape=[
            jax.ShapeDtypeStruct((NPAD, D), jnp.float32),
            jax.ShapeDtypeStruct((NPAD, 128), jnp.float32),
            jax.ShapeDtypeStruct((NREL, 128), jnp.float32),
        ],
    )(x_pad, W, A, R, Ar)


def _ls_body(x_ref, o_ref):
    y = x_ref[...]
    m = jnp.max(y, axis=1, keepdims=True)
    z = y - m
    lse = jnp.log(jnp.sum(jnp.exp(z), axis=1, keepdims=True))
    o_ref[...] = z - lse


def _log_softmax(x):
    BM = 1024
    return pl.pallas_call(
        _ls_body,
        grid=(NPAD // BM,),
        in_specs=[pl.BlockSpec((BM, D), lambda i: (i, 0))],
        out_specs=pl.BlockSpec((BM, D), lambda i: (i, 0)),
        out_shape=jax.ShapeDtypeStruct((NPAD, D), jnp.float32),
    )(x)


def _make_sc_layer(do_relu):
    mesh = plsc.VectorSubcoreMesh(core_axis_name="c", subcore_axis_name="s")

    @functools.partial(
        pl.kernel,
        out_type=[jax.ShapeDtypeStruct((NPAD, D), jnp.float32),
                  jax.ShapeDtypeStruct((E,), jnp.float32)],
        mesh=mesh,
        scratch_types=[
            pltpu.VMEM((NPAD,), jnp.float32),
            pltpu.VMEM((NPAD,), jnp.float32),
            pltpu.VMEM((NREL,), jnp.float32),
            pltpu.VMEM((NREL, D), jnp.float32),
            pltpu.VMEM((D,), jnp.float32),
            pltpu.VMEM((CH,), jnp.int32),
            pltpu.VMEM((CH,), jnp.int32),
            pltpu.VMEM((CH,), jnp.float32),
            pltpu.VMEM((CH,), jnp.float32),
            pltpu.VMEM((CH,), jnp.float32),
            pltpu.VMEM((CH,), jnp.int32),
            pltpu.VMEM((QCAP,), jnp.int32),
            pltpu.VMEM((QCAP,), jnp.float32),
            pltpu.VMEM((KB,), jnp.int32),
            pltpu.VMEM((KB,), jnp.int32),
            pltpu.VMEM((2 * KB,), jnp.int32),
            pltpu.VMEM((2 * KB,), jnp.int32),
            pltpu.VMEM((2 * KB,), jnp.float32),
            pltpu.VMEM((KB, D), jnp.float32),
            pltpu.VMEM((KB, D), jnp.float32),
            pltpu.VMEM((ROWS, D), jnp.float32),
            pltpu.VMEM((ROWS,), jnp.float32),
            pltpu.VMEM((D,), jnp.float32),
            pltpu.VMEM_SHARED((NPAD,), jnp.float32),
            pltpu.SemaphoreType.DMA,
            pltpu.SemaphoreType.DMA,
            pltpu.SemaphoreType.DMA,
            pltpu.SemaphoreType.DMA,
        ],
        compiler_params=pltpu.CompilerParams(needs_layout_passes=False),
    )
    def sc_layer(hs_hbm, hd_hbm, ra_hbm, R_hbm, epk_hbm, bias_hbm, h_hbm,
                 out_hbm, p_hbm,
                 hs_v, hd_v, ra_v, R_v, bias_v, epk_a, epk_b, p_ca, p_cb,
                 p_b, d_b, qmeta, qa, qs_a, qs_b2, qt_b, qd_b, qa_b,
                 hrow_a, hrow_b, out_blk, den_v,
                 zflat, den_spmem, sem_ca, sem_cb, sem_ga, sem_gb):
        cid = lax.axis_index("c")
        sid = lax.axis_index("s")
        wid = cid * NS + sid
        lo = wid * ROWS

        zero_f = jnp.zeros((L,), jnp.float32)
        iota = lax.iota(jnp.int32, L)

        pltpu.sync_copy(hs_hbm, hs_v)
        pltpu.sync_copy(hd_hbm, hd_v)
        pltpu.sync_copy(ra_hbm, ra_v)
        pltpu.sync_copy(R_hbm, R_v)
        pltpu.sync_copy(bias_hbm, bias_v)

        for c in range(D // L):
            zflat[pl.ds(c * L, L)] = zero_f

        def z_body(r, _):
            for c in range(D // L):
                out_blk[r, pl.ds(c * L, L)] = zero_f
            return _

        lax.fori_loop(0, ROWS, z_body, 0)

        @pl.when(sid == 0)
        def _zero_den():
            for j in range(NPAD // D):
                pltpu.sync_copy(zflat, den_spmem.at[pl.ds(j * D, D)])

        plsc.subcore_barrier()

        nch1 = (NCHUNK // NS) + jnp.where(sid < (NCHUNK % NS), 1, 0)

        def p1_chunk(j, _):
            c = sid + j * NS
            pltpu.sync_copy(epk_hbm.at[pl.ds(c * CH, CH)], epk_a)

            def p1_it(i, __):
                w = epk_a[pl.ds(i * L, L)]
                s = w & 0x3FFF
                d = (w >> 14) & 0x3FFF
                t = (w >> 28) & 0xF
                lg = (plsc.load_gather(hs_v, [s])
                      + plsc.load_gather(hd_v, [d])
                      + plsc.load_gather(ra_v, [t]))
                lg = jnp.maximum(lg, 0.2 * lg)
                p = jnp.exp(jnp.minimum(lg, 60.0))
                p_b[pl.ds(i * L, L)] = p
                d_b[pl.ds(i * L, L)] = d
                return __

            lax.fori_loop(0, CH // L, p1_it, 0, unroll=2)
            pltpu.sync_copy(p_b, den_spmem.at[d_b], add=True)
            pltpu.sync_copy(p_b, p_hbm.at[pl.ds(c * CH, CH)])
            return _

        lax.fori_loop(0, nch1, p1_chunk, 0)

        plsc.subcore_barrier()

        pltpu.sync_copy(den_spmem.at[pl.ds(lo, ROWS)], den_v)

        def prep_idx(b, qs_x):
            meta = qmeta[pl.ds(b * KB, L)]
            qs_x[...] = meta & 0x3FFF

        def start_gather(qs_x, hrow_x, sem_x):
            pltpu.async_copy(h_hbm.at[qs_x], hrow_x, sem_x)

        def wait_gather(qs_x, hrow_x, sem_x):
            pltpu.make_async_copy(h_hbm.at[qs_x], hrow_x, sem_x).wait()

        def compute_batch(b, hrow_x):
            off = b * KB
            meta = qmeta[pl.ds(off, L)]
            qt_b[pl.ds(0, L)] = (meta >> 14) & 0xF
            qd_b[pl.ds(0, L)] = meta >> 18
            qa_b[pl.ds(0, L)] = qa[pl.ds(off, L)]

            def ej(j, _):
                aj = qa_b[pl.ds(j, L)][0]
                tj = qt_b[pl.ds(j, L)][0]
                dj = qd_b[pl.ds(j, L)][0]
                for c in range(D // L):
                    plsc.addupdate(
                        out_blk.at[dj, pl.ds(c * L, L)],
                        aj * hrow_x[j, pl.ds(c * L, L)]
                        * R_v[tj, pl.ds(c * L, L)])
                return _

            lax.fori_loop(0, KB, ej, 0, unroll=2)

        def start_first_batches(nf):
            @pl.when(nf > 0)
            def _():
                prep_idx(0, qs_a)
                start_gather(qs_a, hrow_a, sem_ga)

            @pl.when(nf > 1)
            def _():
                prep_idx(1, qs_b2)
                start_gather(qs_b2, hrow_b, sem_gb)

        def drain_batches(nf):
            def pair(j, zz):
                b0 = 2 * j
                b1 = b0 + 1

                @pl.when(b0 < nf)
                def _():
                    wait_gather(qs_a, hrow_a, sem_ga)
                    compute_batch(b0, hrow_a)

                    @pl.when(b0 + 2 < nf)
                    def _():
                        prep_idx(b0 + 2, qs_a)
                        start_gather(qs_a, hrow_a, sem_ga)

                @pl.when(b1 < nf)
                def _():
                    wait_gather(qs_b2, hrow_b, sem_gb)
                    compute_batch(b1, hrow_b)

                    @pl.when(b1 + 2 < nf)
                    def _():
                        prep_idx(b1 + 2, qs_b2)
                        start_gather(qs_b2, hrow_b, sem_gb)

                return zz

            lax.fori_loop(0, (nf + 1) // 2, pair, 0)

        def slide_queue(qp, nf):
            rem = qp - nf * KB

            @pl.when(nf > 0)
            def _():
                def mv(k, _):
                    vm = qmeta[pl.ds(nf * KB + k * L, L)]
                    va = qa[pl.ds(nf * KB + k * L, L)]
                    qmeta[pl.ds(k * L, L)] = vm
                    qa[pl.ds(k * L, L)] = va
                    return _

                lax.fori_loop(0, (rem + L - 1) // L, mv, 0)

            return rem

        def scan_chunk(epk_x, p_cx, qp):
            def p2_it(i, qp):
                w = epk_x[pl.ds(i * L, L)]
                s = w & 0x3FFF
                d = (w >> 14) & 0x3FFF
                t = (w >> 28) & 0xF
                dloc = d - lo
                m = (dloc >= 0) & (dloc < ROWS)
                dlc = jnp.clip(dloc, 0, ROWS - 1)
                p = p_cx[pl.ds(i * L, L)]
                den_g = plsc.load_gather(den_v, [dlc])
                a = p / (den_g + 1e-16)
                pk = s | (t << 14) | (dlc << 18)
                plsc.store_compressed(qmeta.at[pl.ds(qp, L)], pk, mask=m)
                plsc.store_compressed(qa.at[pl.ds(qp, L)], a, mask=m)
                cnt = plsc.all_reduce_population_count(m)
                return qp + cnt[0]

            return lax.fori_loop(0, CH // L, p2_it, qp, unroll=2)

        def start_chunk(c, epk_x, p_cx, sem_x):
            pltpu.async_copy(epk_hbm.at[pl.ds(c * CH, CH)], epk_x, sem_x)
            pltpu.async_copy(p_hbm.at[pl.ds(c * CH, CH)], p_cx, sem_x)

        def wait_chunk(epk_x, p_cx, sem_x):
            pltpu.make_async_copy(epk_hbm.at[pl.ds(0, CH)],
                                  epk_x, sem_x).wait()
            pltpu.make_async_copy(p_hbm.at[pl.ds(0, CH)],
                                  p_cx, sem_x).wait()

        def drain_inflight(np):
            @pl.when(np > 0)
            def _():
                wait_gather(qs_a, hrow_a, sem_ga)
                compute_batch(0, hrow_a)

            @pl.when(np > 1)
            def _():
                wait_gather(qs_b2, hrow_b, sem_gb)
                compute_batch(1, hrow_b)

        def handle_chunk(c, epk_x, p_cx, sem_x, carry):
            qp, np = carry
            wait_chunk(epk_x, p_cx, sem_x)
            qp = scan_chunk(epk_x, p_cx, qp)

            @pl.when(c + 2 < NCHUNK)
            def _():
                start_chunk(c + 2, epk_x, p_cx, sem_x)

            drain_inflight(np)
            qp = slide_queue(qp, np)

            nf_x = jnp.where(qp >= OVFL, qp // KB, 0)
            start_first_batches(nf_x)
            drain_batches(nf_x)
            qp = slide_queue(qp, nf_x)

            np = jnp.minimum(qp // KB, 2)
            start_first_batches(np)
            return qp, np

        start_chunk(0, epk_a, p_ca, sem_ca)
        start_chunk(1, epk_b, p_cb, sem_cb)

        def p2_pair(i, carry):
            carry = handle_chunk(2 * i, epk_a, p_ca, sem_ca, carry)
            carry = handle_chunk(2 * i + 1, epk_b, p_cb, sem_cb, carry)
            return carry

        qp, np = lax.fori_loop(0, NCHUNK // 2, p2_pair, (0, 0))

        drain_inflight(np)
        qp = slide_queue(qp, np)
        qmeta[pl.ds(qp, L)] = jnp.zeros((L,), jnp.int32)
        qa[pl.ds(qp, L)] = zero_f
        nfin = (qp + KB - 1) // KB
        start_first_batches(nfin)
        drain_batches(nfin)

        def o_body(r, _):
            for c in range(D // L):
                v = out_blk[r, pl.ds(c * L, L)] + bias_v[pl.ds(c * L, L)]
                if do_relu:
                    v = jnp.maximum(v, 0.0)
                out_blk[r, pl.ds(c * L, L)] = v
            return _

        lax.fori_loop(0, ROWS, o_body, 0)
        pltpu.sync_copy(out_blk, out_hbm.at[pl.ds(lo, ROWS)])

    return sc_layer


_sc_layer_relu = _make_sc_layer(True)
_sc_layer_plain = _make_sc_layer(False)


def _layer(x_pad, epk, W, R, a_s, a_d, a_r, b, do_relu):
    A = jnp.zeros((D, 128), jnp.float32)
    A = A.at[:, 0].set(a_s).at[:, 1].set(a_d)
    Ar = jnp.zeros((D, 128), jnp.float32).at[:, 0].set(a_r)
    h, aux, raux = _mm(x_pad, W, A, R, Ar)
    hs = aux[:, 0]
    hd = aux[:, 1]
    ra = raux[:NREL, 0]
    fn = _sc_layer_relu if do_relu else _sc_layer_plain
    out, _p = fn(hs, hd, ra, R, epk, b, h)
    return out


def kernel(x, edge, edge_type, W1, R1, a1_src, a1_dst, a1_rel, b1,
           W2, R2, a2_src, a2_dst, a2_rel, b2):
    epk = edge[0] | (edge[1] << 14) | (edge_type << 28)
    x_pad = jnp.zeros((NPAD, D), jnp.float32).at[:N].set(x)
    h1 = _layer(x_pad, epk, W1, R1, a1_src, a1_dst, a1_rel, b1, True)
    h2 = _layer(h1, epk, W2, R2, a2_src, a2_dst, a2_rel, b2, False)
    out = _log_softmax(h2)
    return out[:N]
